# baseline (speedup 1.0000x reference)
import dataclasses

import jax
import jax.numpy as jnp
import numpy as np
from jax import lax
from jax.experimental import pallas as pl
from jax.experimental.pallas import tpu as pltpu
from jax.experimental.pallas import tpu_sc as plsc

D_MODEL = 1024
END_IDX = 2
NW = 32
INF = np.int32(2**30)


def _sc_build_tree(tokens):
    B, T = tokens.shape
    NCHUNK = T // 16
    mesh = plsc.VectorSubcoreMesh(core_axis_name="c", subcore_axis_name="s")
    cp = pltpu.CompilerParams()
    if "needs_layout_passes" in pltpu.CompilerParams.__dataclass_fields__:
        cp = dataclasses.replace(cp, needs_layout_passes=False)

    @pl.kernel(
        compiler_params=cp,
        out_type=jax.ShapeDtypeStruct((B, T // 4, 128), jnp.int32),
        mesh=mesh,
        scratch_types=[
            pltpu.VMEM((T,), jnp.int32),
            pltpu.VMEM((T + 8,), jnp.int32),
            pltpu.VMEM((T,), jnp.int32),
            pltpu.VMEM((T // 4, 128), jnp.int32),
        ],
    )
    def build(tok_hbm, enc_hbm, tok_v, push_v, pp_v, enc_v):
        wid = lax.axis_index("s") * 2 + lax.axis_index("c")

        @pl.when(wid < B)
        def _():
            b = wid
            pltpu.sync_copy(tok_hbm.at[b], tok_v)
            iota = lax.iota(jnp.int32, 16)
            roll_idx = (iota + 15) & 15
            lane0 = iota == 0
            dnums = lax.GatherDimensionNumbers(
                offset_dims=(), collapsed_slice_dims=(0,), start_index_map=(0,))
            zero16 = jnp.zeros((16,), jnp.int32)

            def roll1(w):
                return lax.gather(
                    w, roll_idx[:, None], dnums, slice_sizes=(1,),
                    mode=lax.GatherScatterMode.PROMISE_IN_BOUNDS)

            push_v[pl.ds(0, 16)] = zero16

            def chunk(i, carry):
                cs, cm = carry
                u = 16 * i + iota
                ld = tok_v[pl.ds(16 * i, 16)]
                ne = ((ld != END_IDX) & (u >= 1)).astype(jnp.int32)
                s = plsc.cumsum(ne) + cs
                bv = jnp.where(u >= 1, 2 * s + 1 - u, INF)
                inc = jnp.minimum(-plsc.cummax(-bv), cm)
                ex = jnp.where(lane0, jnp.full((16,), cm), roll1(inc))
                h = (u - 1) + jnp.minimum(0, ex)
                plsc.store_scatter(push_v, [s], u, mask=ne != 0)
                hidx = jnp.maximum(h >> 1, 0)
                par = plsc.load_gather(push_v, [hidx])
                pp_v[pl.ds(16 * i, 16)] = 4 * par + 1 + (h & 1)
                return (cs + jnp.sum(ne), jnp.minimum(cm, jnp.min(bv)))

            lax.fori_loop(0, NCHUNK, chunk, (np.int32(0), INF))

            enc_v[0, pl.ds(0, 16)] = zero16
            enc_v[0, pl.ds(16, 16)] = zero16
            jconst = [jnp.full((16,), j, jnp.int32) for j in range(16)]

            def lane_bcast(vec, j):
                return lax.gather(
                    vec, jconst[j][:, None], dnums, slice_sizes=(1,),
                    mode=lax.GatherScatterMode.PROMISE_IN_BOUNDS)

            def step(t, pp16):
                pair16 = pp16 & 3
                par16 = pp16 >> 2
                prow = ((par16 >> 8) << 6) + (par16 & 63)
                plane = (((par16 >> 6) & 3) << 5) + iota
                w0 = plsc.load_gather(enc_v, [prow, plane])
                w1 = plsc.load_gather(enc_v, [prow, plane + 16])
                r0 = roll1(w0)
                r1 = roll1(w1)
                c0 = jnp.where(lane0, pair16, lax.shift_right_logical(r0, 30))
                c1 = lax.shift_right_logical(jnp.where(lane0, r0, r1), 30)
                trow = ((t >> 8) << 6) + (t & 63)
                tlane = ((t >> 6) & 3) << 5
                enc_v[trow, pl.ds(tlane, 16)] = (w0 << 2) | c0
                enc_v[trow, pl.ds(tlane + 16, 16)] = (w1 << 2) | c1

            pp_c0 = pp_v[pl.ds(0, 16)]
            for j in range(1, 16):
                step(j, lane_bcast(pp_c0, j))

            @pl.loop(1, NCHUNK)
            def outer(i):
                pp_chunk = pp_v[pl.ds(16 * i, 16)]
                for j in range(16):
                    step(16 * i + j, lane_bcast(pp_chunk, j))

            pltpu.sync_copy(enc_v, enc_hbm.at[b])

    return build(tokens)


def _selector_const():
    m = np.arange(128)[:, None]
    k = np.arange(D_MODEL)[None, :]
    s = ((m & 31) == (k >> 5)) & ((m >> 5) == ((k >> 3) & 3))
    return jnp.asarray(s.astype(np.float32), dtype=jnp.bfloat16)


def _tc_decode(enc_g, sel, mask, poly, n_total, row0, carry=None):
    NROW = enc_g.shape[0]
    ROWS = 64

    def body(*refs):
        if carry is None:
            enc_ref, sel_ref, mask_ref, poly_ref, out_ref = refs
        else:
            _, enc_ref, sel_ref, mask_ref, poly_ref, out_ref = refs
        w = enc_ref[...]
        parts = []
        for g in range(4):
            wg = w[:, 32 * g:32 * (g + 1)]
            parts.append(jnp.concatenate(
                [wg & 255, (wg >> 8) & 255, (wg >> 16) & 255,
                 (wg >> 24) & 255], axis=1))
        by = jnp.concatenate(parts, axis=0)
        by = by.astype(jnp.float32).astype(jnp.bfloat16)
        byte = jnp.dot(by, sel_ref[...],
                       preferred_element_type=jnp.float32).astype(jnp.int32)
        out_ref[...] = jnp.where((byte & mask_ref[...]) != 0,
                                 poly_ref[...], jnp.float32(0.0))

    blk0 = row0 // ROWS
    in_specs = [
        pl.BlockSpec((ROWS, 128), lambda i: (i, 0)),
        pl.BlockSpec((128, D_MODEL), lambda i: (0, 0)),
        pl.BlockSpec((1, D_MODEL), lambda i: (0, 0)),
        pl.BlockSpec((1, D_MODEL), lambda i: (0, 0)),
    ]
    args = (enc_g, sel, mask, poly)
    alias = {}
    if carry is not None:
        in_specs = [pl.BlockSpec(memory_space=pl.ANY)] + in_specs
        args = (carry,) + args
        alias = {0: 0}
    return pl.pallas_call(
        body,
        grid=(NROW // ROWS,),
        in_specs=in_specs,
        out_specs=pl.BlockSpec((4 * ROWS, D_MODEL), lambda i: (i + blk0, 0)),
        out_shape=jax.ShapeDtypeStruct((n_total, D_MODEL), jnp.float32),
        input_output_aliases=alias,
    )(*args)


def kernel(tokens, p):
    B, T = tokens.shape
    sel = _selector_const()
    k = np.arange(D_MODEL)
    mask = jnp.asarray((1 << (k & 7)).astype(np.int32)).reshape(1, D_MODEL)
    poly = jnp.power(p[0], jnp.arange(D_MODEL, dtype=jnp.float32)).reshape(1, D_MODEL)
    h = B // 2
    enc1 = _sc_build_tree(tokens[:h]).reshape(h * T // 4, 128)
    enc2 = _sc_build_tree(tokens[h:]).reshape(h * T // 4, 128)
    out = _tc_decode(enc1, sel, mask, poly, B * T, 0)
    out = _tc_decode(enc2, sel, mask, poly, B * T, h * T // 4, carry=out)
    return out.reshape(B, T, D_MODEL)


# speedup vs baseline: 1.3911x; 1.3911x over previous
import dataclasses

import jax
import jax.numpy as jnp
import numpy as np
from jax import lax
from jax.experimental import pallas as pl
from jax.experimental.pallas import tpu as pltpu
from jax.experimental.pallas import tpu_sc as plsc

D_MODEL = 1024
END_IDX = 2
NW = 32
INF = np.int32(2**30)


def _sc_build_tree(tokens):
    B, T = tokens.shape
    NCHUNK = T // 16
    mesh = plsc.VectorSubcoreMesh(core_axis_name="c", subcore_axis_name="s")
    cp = pltpu.CompilerParams()
    if "needs_layout_passes" in pltpu.CompilerParams.__dataclass_fields__:
        cp = dataclasses.replace(cp, needs_layout_passes=False)

    @pl.kernel(
        compiler_params=cp,
        out_type=jax.ShapeDtypeStruct((B, T // 4, 128), jnp.int32),
        mesh=mesh,
        scratch_types=[
            pltpu.VMEM((T,), jnp.int32),
            pltpu.VMEM((T + 8,), jnp.int32),
            pltpu.VMEM((T,), jnp.int32),
            pltpu.VMEM((T // 4, 128), jnp.int32),
        ],
    )
    def build(tok_hbm, enc_hbm, tok_v, push_v, pp_v, enc_v):
        wid = lax.axis_index("s") * 2 + lax.axis_index("c")

        @pl.when(wid < B)
        def _():
            b = wid
            pltpu.sync_copy(tok_hbm.at[b], tok_v)
            iota = lax.iota(jnp.int32, 16)
            roll_idx = (iota + 15) & 15
            lane0 = iota == 0
            dnums = lax.GatherDimensionNumbers(
                offset_dims=(), collapsed_slice_dims=(0,), start_index_map=(0,))
            zero16 = jnp.zeros((16,), jnp.int32)

            def roll1(w):
                return lax.gather(
                    w, roll_idx[:, None], dnums, slice_sizes=(1,),
                    mode=lax.GatherScatterMode.PROMISE_IN_BOUNDS)

            push_v[pl.ds(0, 16)] = zero16

            def chunk(i, carry):
                cs, cm = carry
                u = 16 * i + iota
                ld = tok_v[pl.ds(16 * i, 16)]
                ne = ((ld != END_IDX) & (u >= 1)).astype(jnp.int32)
                s = plsc.cumsum(ne) + cs
                bv = jnp.where(u >= 1, 2 * s + 1 - u, INF)
                inc = jnp.minimum(-plsc.cummax(-bv), cm)
                ex = jnp.where(lane0, jnp.full((16,), cm), roll1(inc))
                h = (u - 1) + jnp.minimum(0, ex)
                plsc.store_scatter(push_v, [s], u, mask=ne != 0)
                hidx = jnp.maximum(h >> 1, 0)
                par = plsc.load_gather(push_v, [hidx])
                pp_v[pl.ds(16 * i, 16)] = 4 * par + 1 + (h & 1)
                return (cs + jnp.sum(ne), jnp.minimum(cm, jnp.min(bv)))

            lax.fori_loop(0, NCHUNK, chunk, (np.int32(0), INF))

            enc_v[0, pl.ds(0, 16)] = zero16
            enc_v[0, pl.ds(16, 16)] = zero16
            jconst = [jnp.full((16,), j, jnp.int32) for j in range(16)]

            def lane_bcast(vec, j):
                return lax.gather(
                    vec, jconst[j][:, None], dnums, slice_sizes=(1,),
                    mode=lax.GatherScatterMode.PROMISE_IN_BOUNDS)

            def step(t, pp16):
                pair16 = pp16 & 3
                par16 = pp16 >> 2
                prow = ((par16 >> 8) << 6) + (par16 & 63)
                plane = (((par16 >> 6) & 3) << 5) + iota
                w0 = plsc.load_gather(enc_v, [prow, plane])
                w1 = plsc.load_gather(enc_v, [prow, plane + 16])
                r0 = roll1(w0)
                r1 = roll1(w1)
                c0 = jnp.where(lane0, pair16, lax.shift_right_logical(r0, 30))
                c1 = lax.shift_right_logical(jnp.where(lane0, r0, r1), 30)
                trow = ((t >> 8) << 6) + (t & 63)
                tlane = ((t >> 6) & 3) << 5
                enc_v[trow, pl.ds(tlane, 16)] = (w0 << 2) | c0
                enc_v[trow, pl.ds(tlane + 16, 16)] = (w1 << 2) | c1

            pp_c0 = pp_v[pl.ds(0, 16)]
            for j in range(1, 16):
                step(j, lane_bcast(pp_c0, j))

            @pl.loop(1, NCHUNK)
            def outer(i):
                pp_chunk = pp_v[pl.ds(16 * i, 16)]
                for j in range(16):
                    step(16 * i + j, lane_bcast(pp_chunk, j))

            pltpu.sync_copy(enc_v, enc_hbm.at[b])

    return build(tokens)


def _selector_const():
    m = np.arange(128)[:, None]
    k = np.arange(D_MODEL)[None, :]
    s = ((m & 31) == (k >> 5)) & ((m >> 5) == ((k >> 3) & 3))
    return jnp.asarray(s.astype(np.float32), dtype=jnp.bfloat16)


def _tc_decode(enc_g, sel, mask, poly):
    NROW = enc_g.shape[0]
    ROWS = 128

    def body(enc_ref, sel_ref, mask_ref, poly_ref, out_ref):
        w = enc_ref[...]
        parts = []
        for half in range(ROWS // 64):
            for g in range(4):
                wg = w[64 * half:64 * (half + 1), 32 * g:32 * (g + 1)]
                parts.append(jnp.concatenate(
                    [wg & 255, (wg >> 8) & 255, (wg >> 16) & 255,
                     (wg >> 24) & 255], axis=1))
        by = jnp.concatenate(parts, axis=0)
        by = by.astype(jnp.float32).astype(jnp.bfloat16)
        byte = jnp.dot(by, sel_ref[...],
                       preferred_element_type=jnp.float32).astype(jnp.int32)
        out_ref[...] = jnp.where((byte & mask_ref[...]) != 0,
                                 poly_ref[...], jnp.float32(0.0))

    return pl.pallas_call(
        body,
        grid=(NROW // ROWS,),
        in_specs=[
            pl.BlockSpec((ROWS, 128), lambda i: (i, 0)),
            pl.BlockSpec((128, D_MODEL), lambda i: (0, 0)),
            pl.BlockSpec((1, D_MODEL), lambda i: (0, 0)),
            pl.BlockSpec((1, D_MODEL), lambda i: (0, 0)),
        ],
        out_specs=pl.BlockSpec((4 * ROWS, D_MODEL), lambda i: (i, 0)),
        out_shape=jax.ShapeDtypeStruct((4 * NROW, D_MODEL), jnp.float32),
    )(enc_g, sel, mask, poly)


def kernel(tokens, p):
    B, T = tokens.shape
    enc = _sc_build_tree(tokens)
    enc_g = enc.reshape(B * T // 4, 128)
    sel = _selector_const()
    k = np.arange(D_MODEL)
    mask = jnp.asarray((1 << (k & 7)).astype(np.int32)).reshape(1, D_MODEL)
    poly = jnp.power(p[0], jnp.arange(D_MODEL, dtype=jnp.float32)).reshape(1, D_MODEL)
    out = _tc_decode(enc_g, sel, mask, poly)
    return out.reshape(B, T, D_MODEL)
